# Initial kernel scaffold; baseline (speedup 1.0000x reference)
import jax
import jax.numpy as jnp
from jax.experimental import pallas as pl


def kernel(tokens, symbolic_emb, emb_table, gate_w, gate_b, w1, b1, w2, b2, ln_g, ln_b):
    raise NotImplementedError("write your pallas kernel here")


# speedup vs baseline: 2.1971x; 2.1971x over previous
import functools
import jax
import jax.numpy as jnp
from jax import lax
from jax.experimental import pallas as pl
from jax.experimental.pallas import tpu as pltpu

TOPK = 2
ROW_TILE = 128


def _gate_body(x_ref, gw_ref, gb_ref, w_ref, i_ref):
    x = x_ref[...]
    logits = jnp.dot(x, gw_ref[...], preferred_element_type=jnp.float32)
    logits = logits + gb_ref[...]
    m = jnp.max(logits, axis=-1, keepdims=True)
    e = jnp.exp(logits - m)
    probs = e / jnp.sum(e, axis=-1, keepdims=True)
    ne = probs.shape[-1]
    lane = lax.broadcasted_iota(jnp.int32, probs.shape, 1)
    v1 = jnp.max(probs, axis=-1, keepdims=True)
    i1 = jnp.min(jnp.where(probs == v1, lane, ne), axis=-1, keepdims=True)
    masked = jnp.where(lane == i1, -jnp.inf, probs)
    v2 = jnp.max(masked, axis=-1, keepdims=True)
    i2 = jnp.min(jnp.where(masked == v2, lane, ne), axis=-1, keepdims=True)
    w_ref[...] = jnp.concatenate([v1, v2], axis=-1)
    i_ref[...] = jnp.concatenate([i1, i2], axis=-1)


def _gate(x, gw, gb):
    s = x.shape[0]
    return pl.pallas_call(
        _gate_body,
        out_shape=(
            jax.ShapeDtypeStruct((s, TOPK), jnp.float32),
            jax.ShapeDtypeStruct((s, TOPK), jnp.int32),
        ),
    )(x, gw, gb.reshape(1, -1))


def _ffn_body(te_ref, xs_ref, w1_ref, b1_ref, w2_ref, b2_ref, sc_ref, out_ref):
    del te_ref
    x = xs_ref[...]
    h = jnp.dot(x, w1_ref[0], preferred_element_type=jnp.float32) + b1_ref[0, 0]
    h = 0.5 * h * (1.0 + lax.erf(h * 0.7071067811865476))
    y = jnp.dot(h, w2_ref[0], preferred_element_type=jnp.float32) + b2_ref[0, 0]
    out_ref[...] = sc_ref[...] * y


def _grouped_ffn(xs, tile_expert, scale, w1, b1, w2, b2):
    np_, d = xs.shape
    ne, _, f = w1.shape
    nt = np_ // ROW_TILE
    grid_spec = pltpu.PrefetchScalarGridSpec(
        num_scalar_prefetch=1,
        grid=(nt,),
        in_specs=[
            pl.BlockSpec((ROW_TILE, d), lambda i, te: (i, 0)),
            pl.BlockSpec((1, d, f), lambda i, te: (te[i], 0, 0)),
            pl.BlockSpec((1, 1, f), lambda i, te: (te[i], 0, 0)),
            pl.BlockSpec((1, f, d), lambda i, te: (te[i], 0, 0)),
            pl.BlockSpec((1, 1, d), lambda i, te: (te[i], 0, 0)),
            pl.BlockSpec((ROW_TILE, 1), lambda i, te: (i, 0)),
        ],
        out_specs=pl.BlockSpec((ROW_TILE, d), lambda i, te: (i, 0)),
    )
    return pl.pallas_call(
        _ffn_body,
        grid_spec=grid_spec,
        out_shape=jax.ShapeDtypeStruct((np_, d), jnp.float32),
    )(tile_expert, xs, w1, b1.reshape(ne, 1, f), w2, b2.reshape(ne, 1, d),
      scale.reshape(np_, 1))


def _ln_body(x_ref, g_ref, b_ref, o_ref):
    x = x_ref[...]
    mu = jnp.mean(x, axis=-1, keepdims=True)
    var = jnp.mean(jnp.square(x - mu), axis=-1, keepdims=True)
    o_ref[...] = (x - mu) * lax.rsqrt(var + 1e-5) * g_ref[...] + b_ref[...]


def _layernorm(x, g, b):
    s, d = x.shape
    return pl.pallas_call(
        _ln_body,
        out_shape=jax.ShapeDtypeStruct((s, d), jnp.float32),
        grid=(s // 256,),
        in_specs=[
            pl.BlockSpec((256, d), lambda i: (i, 0)),
            pl.BlockSpec((1, d), lambda i: (0, 0)),
            pl.BlockSpec((1, d), lambda i: (0, 0)),
        ],
        out_specs=pl.BlockSpec((256, d), lambda i: (i, 0)),
    )(x, g.reshape(1, d), b.reshape(1, d))


def _route(wts, idx, ne):
    s = idx.shape[0]
    na = s * TOPK
    npad = na + ne * ROW_TILE
    iflat = idx.reshape(na)
    wflat = wts.reshape(na)
    oh = (iflat[:, None] == jnp.arange(ne, dtype=jnp.int32)[None, :]).astype(jnp.int32)
    csum = jnp.cumsum(oh, axis=0)
    rank = jnp.take_along_axis(csum, iflat[:, None], axis=1)[:, 0] - 1
    gsz = csum[-1]
    gpad = ((gsz + ROW_TILE - 1) // ROW_TILE) * ROW_TILE
    off = jnp.concatenate([jnp.zeros((1,), jnp.int32), jnp.cumsum(gpad)])
    pos = off[iflat] + rank
    tok = jnp.arange(na, dtype=jnp.int32) // TOPK
    tok_idx = jnp.zeros((npad,), jnp.int32).at[pos].set(tok)
    scale = jnp.zeros((npad,), jnp.float32).at[pos].set(wflat)
    nt = npad // ROW_TILE
    r0 = jnp.arange(nt, dtype=jnp.int32) * ROW_TILE
    te = jnp.sum((r0[:, None] >= off[None, 1:]).astype(jnp.int32), axis=1)
    te = jnp.minimum(te, ne - 1)
    return tok_idx, scale, te, pos.reshape(s, TOPK)


def kernel(tokens, symbolic_emb, emb_table, gate_w, gate_b, w1, b1, w2, b2, ln_g, ln_b):
    b, s = tokens.shape
    d = emb_table.shape[1]
    nl, _, ne = gate_w.shape

    x = jnp.take(emb_table, tokens.reshape(-1).astype(jnp.int32), axis=0)
    x = x + symbolic_emb.reshape(1, d)

    for l in range(nl):
        wts, idx = _gate(x, gate_w[l], gate_b[l])
        tok_idx, scale, te, pos = _route(wts, idx, ne)
        xs = jnp.take(x, tok_idx, axis=0)
        ys = _grouped_ffn(xs, te, scale, w1[l], b1[l], w2[l], b2[l])
        x = jnp.take(ys, pos[:, 0], axis=0) + jnp.take(ys, pos[:, 1], axis=0)

    out = _layernorm(x, ln_g, ln_b)
    return out.reshape(b, s, d)
